# baseline (speedup 1.0000x reference)
import functools

import jax
import jax.numpy as jnp
import numpy as np
from jax.experimental import pallas as pl
from jax.experimental.pallas import tpu as pltpu

_BN = 4000
_WSZ = 128
_NWIN = 9
_ROWS = 1152


def _pool_body(nf_ref, seg_ref, segrow_ref, deg_ref, col_ref, ones_ref,
               g_ref, awr_ref, w1t_ref, w2t_ref, b1_ref, b2_ref, out_ref,
               sd_acc, gz_ref, w_ref, *, nblocks, b_real):
    i = pl.program_id(0)
    h = nf_ref.shape[1]

    @pl.when(i == 0)
    def _init():
        sd_acc[...] = jnp.zeros_like(sd_acc)
        gz_ref[...] = jnp.zeros_like(gz_ref)
        gz_ref[pl.ds(0, b_real), :] = jax.lax.dot(
            g_ref[...].astype(jnp.bfloat16), awr_ref[...],
            preferred_element_type=jnp.float32).astype(jnp.bfloat16)

    nf = nf_ref[...]
    nf_bf = nf.astype(jnp.bfloat16)
    segf = seg_ref[...]
    segrow = segrow_ref[0]
    deg = deg_ref[...]
    col = col_ref[...]
    ones = ones_ref[...]
    bn = nf.shape[0]

    zraw = jax.lax.dot(nf_bf, awr_ref[...],
                       preferred_element_type=jnp.float32)

    s_first = segf[0, 0]
    s_last = segf[bn - 1, 0]
    s_base = jnp.floor(s_first / 8.0) * 8.0
    s0i = pl.multiple_of(s_base.astype(jnp.int32), 8)

    dcast = jax.lax.dot(segf - s_base, ones,
                        preferred_element_type=jnp.float32)
    deg128 = jax.lax.dot(deg, ones,
                         preferred_element_type=jnp.float32)

    onehot0 = (dcast == col).astype(jnp.bfloat16)
    gzv = jax.lax.dot(onehot0, gz_ref[pl.ds(s0i, _WSZ), :],
                      preferred_element_type=jnp.float32)
    w_ref[...] = jnp.exp((zraw + gzv) * deg128).astype(jnp.bfloat16)

    for w in range(1, _NWIN):
        @pl.when(s_base + w * _WSZ <= s_last)
        def _extra_gather(w=w):
            ohw = (segf - (s_base + w * _WSZ) == col).astype(jnp.bfloat16)
            gzw = jax.lax.dot(ohw, gz_ref[pl.ds(s0i + w * _WSZ, _WSZ), :],
                              preferred_element_type=jnp.float32)
            w_ref[...] *= jnp.exp(gzw * deg128).astype(jnp.bfloat16)

    w128 = w_ref[...]
    u2 = jnp.concatenate([nf_bf * w128, w128], axis=1)

    rowi = jax.lax.broadcasted_iota(jnp.int32, (_WSZ, bn), 0).astype(
        jnp.float32)
    oht = (rowi == segrow - s_base).astype(jnp.bfloat16)
    sd_acc[pl.ds(s0i, _WSZ), :] += jax.lax.dot(
        oht, u2, preferred_element_type=jnp.float32)

    contract = (((0,), (0,)), ((), ()))
    for w in range(1, _NWIN):
        @pl.when(s_base + w * _WSZ <= s_last)
        def _extra_scatter(w=w):
            ohw = (segf - (s_base + w * _WSZ) == col).astype(jnp.bfloat16)
            sd_acc[pl.ds(s0i + w * _WSZ, _WSZ), :] += jax.lax.dot_general(
                ohw, u2, contract, preferred_element_type=jnp.float32)

    @pl.when(i == nblocks - 1)
    def _finish():
        s = sd_acc[pl.ds(0, b_real), pl.ds(0, h)]
        d = sd_acc[pl.ds(0, b_real), pl.ds(h, h)]
        g = g_ref[...]
        he = jnp.where(d > 0.0, s / d + g, 0.0)
        h1 = jax.nn.relu(
            jax.lax.dot(he, w1t_ref[...], preferred_element_type=jnp.float32)
            + b1_ref[...])
        h2 = jax.lax.dot(h1, w2t_ref[...],
                         preferred_element_type=jnp.float32) + b2_ref[...]
        out_ref[...] = h2 + g


def kernel(node_feats, g_feats, degree, segment_ids, attn, W1, b1, W2, b2):
    n, h = node_feats.shape
    b, _ = g_feats.shape
    nh, dh = attn.shape[1], attn.shape[2]

    segf = segment_ids.astype(jnp.float32)
    seg_col = segf.reshape(n, 1)
    seg_row = segf.reshape(n // _BN, 1, _BN)
    col = jnp.arange(_WSZ, dtype=jnp.float32).reshape(1, _WSZ)
    ones = jnp.ones((1, h), jnp.float32)

    headmask = np.kron(np.eye(nh, dtype=np.float32),
                       np.ones((dh, dh), np.float32))
    awr = (attn.reshape(nh * dh, 1) * headmask).astype(jnp.bfloat16)

    nblocks = n // _BN

    body = functools.partial(_pool_body, nblocks=nblocks, b_real=b)
    out = pl.pallas_call(
        body,
        grid=(nblocks,),
        in_specs=[
            pl.BlockSpec((_BN, h), lambda i: (i, 0)),
            pl.BlockSpec((_BN, 1), lambda i: (i, 0)),
            pl.BlockSpec((1, 1, _BN), lambda i: (i, 0, 0)),
            pl.BlockSpec((_BN, 1), lambda i: (i, 0)),
            pl.BlockSpec((1, _WSZ), lambda i: (0, 0)),
            pl.BlockSpec((1, h), lambda i: (0, 0)),
            pl.BlockSpec((b, h), lambda i: (0, 0)),
            pl.BlockSpec((h, h), lambda i: (0, 0)),
            pl.BlockSpec((h, h), lambda i: (0, 0)),
            pl.BlockSpec((h, h), lambda i: (0, 0)),
            pl.BlockSpec((1, h), lambda i: (0, 0)),
            pl.BlockSpec((1, h), lambda i: (0, 0)),
        ],
        out_specs=pl.BlockSpec((b, h), lambda i: (0, 0)),
        out_shape=jax.ShapeDtypeStruct((b, h), jnp.float32),
        scratch_shapes=[
            pltpu.VMEM((_ROWS, 2 * h), jnp.float32),
            pltpu.VMEM((_ROWS, h), jnp.bfloat16),
            pltpu.VMEM((_BN, h), jnp.bfloat16),
        ],
        compiler_params=pltpu.CompilerParams(
            dimension_semantics=("arbitrary",)),
    )(node_feats, seg_col, seg_row, degree, col, ones, g_feats, awr,
      W1.T, W2.T, b1.reshape(1, h), b2.reshape(1, h))
    return out


# speedup vs baseline: 1.0045x; 1.0045x over previous
import functools

import jax
import jax.numpy as jnp
import numpy as np
from jax.experimental import pallas as pl
from jax.experimental.pallas import tpu as pltpu

_BN = 4000
_WSZ = 128
_NWIN = 9
_ROWS = 1152


def _pool_body(nf_ref, seg_ref, segrow_ref, deg_ref, col_ref, ones_ref,
               g_ref, awr_ref, w1t_ref, w2t_ref, b1_ref, b2_ref, out_ref,
               sd_acc, gz_ref, w_ref, *, nblocks, b_real):
    i = pl.program_id(0)
    h = nf_ref.shape[1]

    @pl.when(i == 0)
    def _init():
        sd_acc[...] = jnp.zeros_like(sd_acc)
        gz_ref[...] = jnp.zeros_like(gz_ref)
        gz_ref[pl.ds(0, b_real), :] = jax.lax.dot(
            g_ref[...].astype(jnp.bfloat16), awr_ref[...],
            preferred_element_type=jnp.float32).astype(jnp.bfloat16)

    nf = nf_ref[...]
    nf_bf = nf.astype(jnp.bfloat16)
    segf = seg_ref[...]
    segrow = segrow_ref[0]
    deg = deg_ref[...]
    col = col_ref[...]
    ones = ones_ref[...]
    ones_bf = ones.astype(jnp.bfloat16)
    bn = nf.shape[0]

    s_first = segf[0, 0]
    s_last = segf[bn - 1, 0]
    s_base = jnp.floor(s_first / 16.0) * 16.0
    s0i = pl.multiple_of(s_base.astype(jnp.int32), 16)

    dcast = jax.lax.dot((segf - s_base).astype(jnp.bfloat16), ones_bf,
                        preferred_element_type=jnp.float32)
    deg128 = jax.lax.dot(deg, ones,
                         preferred_element_type=jnp.float32)

    onehot0 = (dcast == col).astype(jnp.bfloat16)
    lhs = jnp.concatenate([onehot0, nf_bf], axis=1)
    rhs = jnp.concatenate([gz_ref[pl.ds(s0i, _WSZ), :], awr_ref[...]], axis=0)
    zg = jax.lax.dot(lhs, rhs, preferred_element_type=jnp.float32)
    w_ref[...] = jnp.exp(zg * deg128).astype(jnp.bfloat16)

    for w in range(1, _NWIN):
        @pl.when(s_base + w * _WSZ <= s_last)
        def _extra_gather(w=w):
            ohw = (segf - (s_base + w * _WSZ) == col).astype(jnp.bfloat16)
            gzw = jax.lax.dot(ohw, gz_ref[pl.ds(s0i + w * _WSZ, _WSZ), :],
                              preferred_element_type=jnp.float32)
            w_ref[...] *= jnp.exp(gzw * deg128).astype(jnp.bfloat16)

    w128 = w_ref[...]
    u2 = jnp.concatenate([nf_bf * w128, w128], axis=1)

    rowi = jax.lax.broadcasted_iota(jnp.int32, (_WSZ, bn), 0).astype(
        jnp.float32)
    oht = (rowi == segrow - s_base).astype(jnp.bfloat16)
    sd_acc[pl.ds(s0i, _WSZ), :] += jax.lax.dot(
        oht, u2, preferred_element_type=jnp.float32)

    contract = (((0,), (0,)), ((), ()))
    for w in range(1, _NWIN):
        @pl.when(s_base + w * _WSZ <= s_last)
        def _extra_scatter(w=w):
            ohw = (segf - (s_base + w * _WSZ) == col).astype(jnp.bfloat16)
            sd_acc[pl.ds(s0i + w * _WSZ, _WSZ), :] += jax.lax.dot_general(
                ohw, u2, contract, preferred_element_type=jnp.float32)

    @pl.when(i == nblocks - 1)
    def _finish():
        s = sd_acc[pl.ds(0, b_real), pl.ds(0, h)]
        d = sd_acc[pl.ds(0, b_real), pl.ds(h, h)]
        g = g_ref[...]
        he = jnp.where(d > 0.0, s / d + g, 0.0)
        h1 = jax.nn.relu(
            jax.lax.dot(he, w1t_ref[...], preferred_element_type=jnp.float32)
            + b1_ref[...])
        h2 = jax.lax.dot(h1, w2t_ref[...],
                         preferred_element_type=jnp.float32) + b2_ref[...]
        out_ref[...] = h2 + g


def kernel(node_feats, g_feats, degree, segment_ids, attn, W1, b1, W2, b2):
    n, h = node_feats.shape
    b, _ = g_feats.shape
    nh, dh = attn.shape[1], attn.shape[2]

    segf = segment_ids.astype(jnp.float32)
    seg_col = segf.reshape(n, 1)
    seg_row = segf.reshape(n // _BN, 1, _BN)
    col = jnp.arange(_WSZ, dtype=jnp.float32).reshape(1, _WSZ)
    ones = jnp.ones((1, h), jnp.float32)

    headmask = np.kron(np.eye(nh, dtype=np.float32),
                       np.ones((dh, dh), np.float32))
    awr = (attn.reshape(nh * dh, 1) * headmask).astype(jnp.bfloat16)

    nblocks = n // _BN

    body = functools.partial(_pool_body, nblocks=nblocks, b_real=b)
    out = pl.pallas_call(
        body,
        grid=(nblocks,),
        in_specs=[
            pl.BlockSpec((_BN, h), lambda i: (i, 0)),
            pl.BlockSpec((_BN, 1), lambda i: (i, 0)),
            pl.BlockSpec((1, 1, _BN), lambda i: (i, 0, 0)),
            pl.BlockSpec((_BN, 1), lambda i: (i, 0)),
            pl.BlockSpec((1, _WSZ), lambda i: (0, 0)),
            pl.BlockSpec((1, h), lambda i: (0, 0)),
            pl.BlockSpec((b, h), lambda i: (0, 0)),
            pl.BlockSpec((h, h), lambda i: (0, 0)),
            pl.BlockSpec((h, h), lambda i: (0, 0)),
            pl.BlockSpec((h, h), lambda i: (0, 0)),
            pl.BlockSpec((1, h), lambda i: (0, 0)),
            pl.BlockSpec((1, h), lambda i: (0, 0)),
        ],
        out_specs=pl.BlockSpec((b, h), lambda i: (0, 0)),
        out_shape=jax.ShapeDtypeStruct((b, h), jnp.float32),
        scratch_shapes=[
            pltpu.VMEM((_ROWS, 2 * h), jnp.float32),
            pltpu.VMEM((_ROWS, h), jnp.bfloat16),
            pltpu.VMEM((_BN, h), jnp.bfloat16),
        ],
        compiler_params=pltpu.CompilerParams(
            dimension_semantics=("arbitrary",)),
    )(node_feats, seg_col, seg_row, degree, col, ones, g_feats, awr,
      W1.T, W2.T, b1.reshape(1, h), b2.reshape(1, h))
    return out
